# baseline (speedup 1.0000x reference)
import jax
import jax.numpy as jnp
from jax.experimental import pallas as pl
from jax.experimental.pallas import tpu as pltpu

TILE = 256


def _make_body(n, s_total, num_tiles, num_steps, tile):
    def body(types_ref, h_ref, m_ref, w2_ref, o_ref, bounds, w_ref):
        w_ref[...] = w2_ref[...].reshape(w_ref.shape)
        types = types_ref[...]

        def count(s, carry):
            bounds[s] = jnp.sum((types < s).astype(jnp.int32))
            return carry

        jax.lax.fori_loop(0, s_total + 1, count, 0)

        iota = jax.lax.broadcasted_iota(jnp.int32, (tile, 1), 0)

        def step(g, carry):
            t, s, first, acc = carry
            done = t >= num_tiles
            tc = jnp.minimum(t, num_tiles - 1)
            sc = jnp.minimum(s, s_total - 1)
            tile_start = tc * tile
            tile_end = jnp.minimum(tile_start + tile, n)
            r0 = pl.multiple_of(jnp.minimum(tile_start, n - tile), 8)
            row_lo = jnp.maximum(bounds[sc], tile_start)
            row_hi = jnp.where(done, 0, bounds[sc + 1])
            rows = r0 + iota
            mask = (rows >= row_lo) & (rows < row_hi)
            xm = jnp.where(mask, m_ref[pl.ds(r0, tile), :], 0.0)
            mm = jax.lax.dot_general(
                xm, w_ref[sc],
                (((1,), (1,)), ((), ())),
                preferred_element_type=jnp.float32,
            )
            acc = jnp.where(first, h_ref[pl.ds(r0, tile), :], acc) + mm

            seg_end = bounds[sc + 1]
            not_done = jnp.logical_not(done)
            adv_t = jnp.logical_and(not_done, seg_end >= tile_end)
            s_next = s + jnp.logical_and(not_done, seg_end <= tile_end)
            t_next = t + adv_t

            @pl.when(jnp.logical_and(adv_t, tc < num_tiles - 1))
            def _():
                o_ref[pl.ds(r0, tile), :] = acc

            @pl.when(jnp.logical_and(adv_t, tc == num_tiles - 1))
            def _():
                o_ref[pl.ds(r0, tile), :] = jnp.where(
                    rows >= tile_start, acc, o_ref[pl.ds(r0, tile), :]
                )

            return t_next, s_next, adv_t, acc

        acc0 = jnp.zeros((tile, h_ref.shape[1]), jnp.float32)
        jax.lax.fori_loop(
            0, num_steps, step,
            (jnp.int32(0), jnp.int32(0), jnp.bool_(True), acc0),
        )

    return body


@jax.jit
def kernel(h_prev, m_curr, atom_types, weight):
    n, h = h_prev.shape
    s = weight.shape[0]
    num_tiles = pl.cdiv(n, TILE)
    num_steps = num_tiles + s - 1

    vmem = pl.BlockSpec(memory_space=pltpu.VMEM)
    out = pl.pallas_call(
        _make_body(n, s, num_tiles, num_steps, TILE),
        in_specs=[vmem, vmem, vmem, vmem],
        out_specs=vmem,
        out_shape=jax.ShapeDtypeStruct((n, h), jnp.float32),
        scratch_shapes=[pltpu.SMEM((s + 1,), jnp.int32),
                        pltpu.VMEM((s, h, h), jnp.float32)],
    )(atom_types.astype(jnp.int32), h_prev, m_curr, weight)
    return out


# speedup vs baseline: 1.3566x; 1.3566x over previous
import jax
import jax.numpy as jnp
from jax.experimental import pallas as pl
from jax.experimental.pallas import tpu as pltpu

TILE = 256


def _make_body(n, s_total, num_tiles, num_steps, tile):
    n_groups = (s_total + 8) // 8
    n_chunks = pl.cdiv(n, 128)

    def body(types_ref, h_ref, m_ref, w2_ref, o_ref,
             bounds, w_ref, m_bf_ref, bvec, tpad_ref, dma_sem):
        w_ref[...] = w2_ref[...].reshape(w_ref.shape).astype(jnp.bfloat16)
        m_bf_ref[...] = m_ref[...].astype(jnp.bfloat16)

        sub = jax.lax.broadcasted_iota(jnp.int32, (8, 1), 0)
        tpad_ref[pl.ds(128 * (n_chunks - 1), 128)] = jnp.full(
            (128,), jnp.int32(2 ** 30))
        tpad_ref[pl.ds(0, n)] = types_ref[...]

        def count_chunk(c, accs):
            chunk = tpad_ref[pl.ds(c * 128, 128)]
            x = jnp.broadcast_to(chunk[None, :], (8, 128))
            return tuple(
                accs[g] + (x < (8 * g + sub)).astype(jnp.int32)
                for g in range(n_groups)
            )

        accs = jax.lax.fori_loop(
            0, n_chunks, count_chunk,
            tuple(jnp.zeros((8, 128), jnp.int32) for _ in range(n_groups)),
        )
        for g in range(n_groups):
            bvec[pl.ds(8 * g, 8), :] = jnp.sum(accs[g], axis=1, keepdims=True)
        copy = pltpu.make_async_copy(bvec, bounds, dma_sem)
        copy.start()
        copy.wait()

        iota = jax.lax.broadcasted_iota(jnp.int32, (tile, 1), 0)

        def step(g, carry):
            t, s, first, acc = carry
            done = t >= num_tiles
            tc = jnp.minimum(t, num_tiles - 1)
            sc = jnp.minimum(s, s_total - 1)
            tile_start = tc * tile
            tile_end = jnp.minimum(tile_start + tile, n)
            r0 = pl.multiple_of(jnp.minimum(tile_start, n - tile), 8)
            row_lo = jnp.maximum(bounds[sc, 0], tile_start)
            row_hi = jnp.where(done, 0, bounds[sc + 1, 0])
            rows = r0 + iota
            mask = (rows >= row_lo) & (rows < row_hi)
            xm = jnp.where(mask, m_bf_ref[pl.ds(r0, tile), :],
                           jnp.bfloat16(0.0))
            mm = jax.lax.dot_general(
                xm, w_ref[sc],
                (((1,), (1,)), ((), ())),
                preferred_element_type=jnp.float32,
            )
            acc = jnp.where(first, h_ref[pl.ds(r0, tile), :], acc) + mm

            seg_end = bounds[sc + 1, 0]
            not_done = jnp.logical_not(done)
            adv_t = jnp.logical_and(not_done, seg_end >= tile_end)
            s_next = s + jnp.logical_and(not_done, seg_end <= tile_end)
            t_next = t + adv_t

            @pl.when(jnp.logical_and(adv_t, tc < num_tiles - 1))
            def _():
                o_ref[pl.ds(r0, tile), :] = acc

            @pl.when(jnp.logical_and(adv_t, tc == num_tiles - 1))
            def _():
                o_ref[pl.ds(r0, tile), :] = jnp.where(
                    rows >= tile_start, acc, o_ref[pl.ds(r0, tile), :]
                )

            return t_next, s_next, adv_t, acc

        acc0 = jnp.zeros((tile, h_ref.shape[1]), jnp.float32)
        jax.lax.fori_loop(
            0, num_steps, step,
            (jnp.int32(0), jnp.int32(0), jnp.bool_(True), acc0),
        )

    return body


@jax.jit
def kernel(h_prev, m_curr, atom_types, weight):
    n, h = h_prev.shape
    s = weight.shape[0]
    num_tiles = pl.cdiv(n, TILE)
    num_steps = num_tiles + s - 1
    n_groups = (s + 8) // 8

    vmem = pl.BlockSpec(memory_space=pltpu.VMEM)
    out = pl.pallas_call(
        _make_body(n, s, num_tiles, num_steps, TILE),
        in_specs=[vmem, vmem, vmem, vmem],
        out_specs=vmem,
        out_shape=jax.ShapeDtypeStruct((n, h), jnp.float32),
        scratch_shapes=[
            pltpu.SMEM((8 * n_groups, 1), jnp.int32),
            pltpu.VMEM((s, h, h), jnp.bfloat16),
            pltpu.VMEM((n, h), jnp.bfloat16),
            pltpu.VMEM((8 * n_groups, 1), jnp.int32),
            pltpu.VMEM((128 * pl.cdiv(n, 128),), jnp.int32),
            pltpu.SemaphoreType.DMA,
        ],
    )(atom_types.astype(jnp.int32), h_prev, m_curr, weight)
    return out
